# baseline (speedup 1.0000x reference)
import functools

import jax
import jax.numpy as jnp
from jax import lax
from jax.experimental import pallas as pl
from jax.experimental.pallas import tpu as pltpu
from jax.experimental.pallas import tpu_sc as plsc

B_TOK = 16384
S_TOK = 26
ROWS = B_TOK * S_TOK
VOCAB = 1000000
DIM = 64
CHUNK = 128
NBUF = 3
LAG = 2
TBLK = 256
NFULL = VOCAB // TBLK
TAIL = VOCAB - NFULL * TBLK

_PARAMS = pltpu.CompilerParams(
    use_tc_tiling_on_sc=False, needs_layout_passes=False)


def _sc_geometry():
    try:
        info = plsc.get_sparse_core_info()
        return info.num_cores, info.num_subcores
    except Exception:
        return 2, 16


def _make_transpose(num_cores, num_subcores):
    nw = num_cores * num_subcores

    mesh = plsc.VectorSubcoreMesh(
        core_axis_name="c",
        subcore_axis_name="s",
        num_cores=num_cores,
        num_subcores=num_subcores,
    )

    @functools.partial(
        pl.kernel,
        out_type=jax.ShapeDtypeStruct((VOCAB, DIM), jnp.float32),
        mesh=mesh,
        scratch_types=[
            pltpu.VMEM((2, DIM, TBLK), jnp.float32),
            pltpu.VMEM((2, TBLK, DIM), jnp.float32),
            pltpu.SemaphoreType.DMA((2,)),
            pltpu.SemaphoreType.DMA((2,)),
        ],
        compiler_params=_PARAMS,
    )
    def transpose_kernel(wt_t, wt_rm, in_v, out_v, isem, wsem):
        wid = lax.axis_index("s") * num_cores + lax.axis_index("c")
        nblk_w = (NFULL - wid + nw - 1) // nw
        iota16 = lax.iota(jnp.int32, 16)

        def col0(blk):
            return (wid + blk * nw) * TBLK

        def start_in(blk, buf):
            pltpu.async_copy(
                wt_t.at[:, pl.ds(col0(blk), TBLK)],
                in_v.at[buf], isem.at[buf])

        def wait_in(blk, buf):
            pltpu.make_async_copy(
                wt_t.at[:, pl.ds(col0(blk), TBLK)],
                in_v.at[buf], isem.at[buf]).wait()

        def start_write(blk, buf):
            pltpu.async_copy(
                out_v.at[buf],
                wt_rm.at[pl.ds(col0(blk), TBLK), :],
                wsem.at[buf])

        def wait_write(blk, buf):
            pltpu.make_async_copy(
                out_v.at[buf],
                wt_rm.at[pl.ds(col0(blk), TBLK), :],
                wsem.at[buf]).wait()

        def transpose_cols(buf, ncols):
            def body(p, carry):
                pvec = jnp.broadcast_to(p, (16,)).astype(jnp.int32)
                for k in range(DIM // 16):
                    fvec = iota16 + (k * 16)
                    vals = plsc.load_gather(in_v.at[buf], [fvec, pvec])
                    out_v[buf, p, pl.ds(k * 16, 16)] = vals
                return carry

            lax.fori_loop(0, ncols, body, 0)

        @pl.when(nblk_w >= 1)
        def _():
            start_in(0, 0)

        @pl.when(nblk_w >= 2)
        def _():
            start_in(1, 1)

        def step(i, carry):
            blk2 = i * 2
            for b in range(2):
                blk = blk2 + b

                @pl.when(blk < nblk_w)
                def _():
                    wait_in(blk, b)

                    @pl.when(blk >= 2)
                    def _():
                        wait_write(blk - 2, b)

                    transpose_cols(b, TBLK)
                    start_write(blk, b)

                    @pl.when(blk + 2 < nblk_w)
                    def _():
                        start_in(blk + 2, b)

            return carry

        lax.fori_loop(0, (nblk_w + 1) // 2, step, 0)

        @pl.when(nblk_w >= 2)
        def _():
            wait_write(nblk_w - 2, (nblk_w - 2) % 2)

        @pl.when(nblk_w >= 1)
        def _():
            wait_write(nblk_w - 1, (nblk_w - 1) % 2)

        @pl.when(wid == 0)
        def _():
            pltpu.sync_copy(
                wt_t.at[:, pl.ds(NFULL * TBLK, TAIL)],
                in_v.at[0, :, pl.ds(0, TAIL)])
            transpose_cols(0, TAIL)
            pltpu.sync_copy(
                out_v.at[0, pl.ds(0, TAIL), :],
                wt_rm.at[pl.ds(NFULL * TBLK, TAIL), :])

    return transpose_kernel


def _make_gather(num_cores, num_subcores):
    nw = num_cores * num_subcores
    total_chunks = ROWS // CHUNK
    chunks_per_w = total_chunks // nw
    chunks_per_s = B_TOK // CHUNK

    mesh = plsc.VectorSubcoreMesh(
        core_axis_name="c",
        subcore_axis_name="s",
        num_cores=num_cores,
        num_subcores=num_subcores,
    )

    @functools.partial(
        pl.kernel,
        out_type=jax.ShapeDtypeStruct((S_TOK, B_TOK, DIM), jnp.float32),
        mesh=mesh,
        scratch_types=[
            pltpu.VMEM((NBUF, CHUNK), jnp.int32),
            pltpu.VMEM((NBUF, CHUNK, DIM), jnp.float32),
            pltpu.VMEM((NBUF, DIM // 8, 8, CHUNK), jnp.float32),
            pltpu.SemaphoreType.DMA((NBUF,)),
            pltpu.SemaphoreType.DMA((NBUF,)),
            pltpu.SemaphoreType.DMA((NBUF,)),
        ],
        compiler_params=_PARAMS,
    )
    def gather_kernel(idx_hbm, table_hbm, out_hbm, idx_v, rows_v, tr_v,
                      isem, gsem, wsem):
        wid = lax.axis_index("s") * num_cores + lax.axis_index("c")
        chunk0 = wid * chunks_per_w
        iota16 = lax.iota(jnp.int32, 16)

        def coords(chunk_j):
            cg = chunk0 + chunk_j
            return cg // chunks_per_s, cg % chunks_per_s

        def idx_src(chunk_j):
            s_id, bj = coords(chunk_j)
            return idx_hbm.at[s_id, pl.ds(bj * CHUNK, CHUNK)]

        def start_idx(chunk_j, buf):
            pltpu.async_copy(idx_src(chunk_j), idx_v.at[buf], isem.at[buf])

        def wait_idx(chunk_j, buf):
            pltpu.make_async_copy(
                idx_src(chunk_j), idx_v.at[buf], isem.at[buf]).wait()

        def start_gather(buf):
            pltpu.async_copy(
                table_hbm.at[idx_v.at[buf]], rows_v.at[buf], gsem.at[buf])

        def wait_gather(buf):
            pltpu.make_async_copy(
                table_hbm.at[idx_v.at[buf]], rows_v.at[buf],
                gsem.at[buf]).wait()

        def out_dst(chunk_j):
            s_id, bj = coords(chunk_j)
            return out_hbm.at[s_id, pl.ds(bj * CHUNK, CHUNK), :]

        def start_write(chunk_j, buf):
            pltpu.async_copy(rows_v.at[buf], out_dst(chunk_j), wsem.at[buf])

        def wait_write(chunk_j, buf):
            pltpu.make_async_copy(
                rows_v.at[buf], out_dst(chunk_j), wsem.at[buf]).wait()

        def transpose_chunk(buf):
            def body(g, carry):
                cvec = g * 16 + iota16
                for d in range(DIM):
                    dvec = jnp.broadcast_to(jnp.int32(d), (16,))
                    vals = plsc.load_gather(rows_v.at[buf], [cvec, dvec])
                    tr_v[buf, d // 8, d % 8, pl.ds(g * 16, 16)] = vals
                return carry

            lax.fori_loop(0, CHUNK // 16, body, 0)

        for b in range(NBUF):
            start_idx(b, b)

        def step(i, carry):
            cbase = i * NBUF
            for b in range(NBUF):
                c = cbase + b

                @pl.when(c < chunks_per_w)
                def _():
                    wait_idx(c, b)

                    @pl.when(c >= NBUF)
                    def _():
                        wait_write(c - NBUF, b)

                    start_gather(b)

                r = c - LAG
                br = (b - LAG) % NBUF

                @pl.when((r >= 0) & (r < chunks_per_w))
                def _():
                    wait_gather(br)
                    transpose_chunk(br)
                    start_write(r, br)

                    @pl.when(r + NBUF < chunks_per_w)
                    def _():
                        start_idx(r + NBUF, br)

            return carry

        lax.fori_loop(0, (chunks_per_w + LAG + NBUF - 1) // NBUF, step, 0)

        for k in range(NBUF):
            r = chunks_per_w - NBUF + k
            wait_write(r, r % NBUF)

    return gather_kernel


def kernel(token_ids, weight):
    num_cores, num_subcores = _sc_geometry()
    wt_rm = _make_transpose(num_cores, num_subcores)(weight.T)
    idx_t = token_ids.T.astype(jnp.int32)
    out3 = _make_gather(num_cores, num_subcores)(idx_t, wt_rm)
    return out3.transpose(1, 0, 2)


# speedup vs baseline: 5.7262x; 5.7262x over previous
import functools

import jax
import jax.numpy as jnp
from jax import lax
from jax.experimental import pallas as pl
from jax.experimental.pallas import tpu as pltpu
from jax.experimental.pallas import tpu_sc as plsc

B_TOK = 16384
S_TOK = 26
ROWS = B_TOK * S_TOK
VOCAB = 1000000
DIM = 64
CHUNK = 128
NBUF = 3
LAG = 2
TBLK = 256
NFULL = VOCAB // TBLK
TAIL = VOCAB - NFULL * TBLK

_PARAMS = pltpu.CompilerParams(
    use_tc_tiling_on_sc=False, needs_layout_passes=False)


def _sc_geometry():
    try:
        info = plsc.get_sparse_core_info()
        return info.num_cores, info.num_subcores
    except Exception:
        return 2, 16


def _make_transpose(num_cores, num_subcores):
    nw = num_cores * num_subcores

    mesh = plsc.VectorSubcoreMesh(
        core_axis_name="c",
        subcore_axis_name="s",
        num_cores=num_cores,
        num_subcores=num_subcores,
    )

    @functools.partial(
        pl.kernel,
        out_type=jax.ShapeDtypeStruct((VOCAB, DIM), jnp.float32),
        mesh=mesh,
        scratch_types=[
            pltpu.VMEM((2, DIM, TBLK), jnp.float32),
            pltpu.VMEM((2, TBLK, DIM), jnp.float32),
            pltpu.SemaphoreType.DMA((2,)),
            pltpu.SemaphoreType.DMA((2,)),
        ],
        compiler_params=_PARAMS,
    )
    def transpose_kernel(wt_t, wt_rm, in_v, out_v, isem, wsem):
        wid = lax.axis_index("s") * num_cores + lax.axis_index("c")
        nblk_w = (NFULL - wid + nw - 1) // nw
        iota16 = lax.iota(jnp.int32, 16)

        def col0(blk):
            return (wid + blk * nw) * TBLK

        def start_in(blk, buf):
            pltpu.async_copy(
                wt_t.at[:, pl.ds(col0(blk), TBLK)],
                in_v.at[buf], isem.at[buf])

        def wait_in(blk, buf):
            pltpu.make_async_copy(
                wt_t.at[:, pl.ds(col0(blk), TBLK)],
                in_v.at[buf], isem.at[buf]).wait()

        def start_write(blk, buf):
            pltpu.async_copy(
                out_v.at[buf],
                wt_rm.at[pl.ds(col0(blk), TBLK), :],
                wsem.at[buf])

        def wait_write(blk, buf):
            pltpu.make_async_copy(
                out_v.at[buf],
                wt_rm.at[pl.ds(col0(blk), TBLK), :],
                wsem.at[buf]).wait()

        def transpose_cols(buf, ncols):
            def body(p, carry):
                pvec = jnp.broadcast_to(p, (16,)).astype(jnp.int32)
                for k in range(DIM // 16):
                    fvec = iota16 + (k * 16)
                    vals = plsc.load_gather(in_v.at[buf], [fvec, pvec])
                    out_v[buf, p, pl.ds(k * 16, 16)] = vals
                return carry

            lax.fori_loop(0, ncols, body, 0)

        @pl.when(nblk_w >= 1)
        def _():
            start_in(0, 0)

        @pl.when(nblk_w >= 2)
        def _():
            start_in(1, 1)

        def step(i, carry):
            blk2 = i * 2
            for b in range(2):
                blk = blk2 + b

                @pl.when(blk < nblk_w)
                def _():
                    wait_in(blk, b)

                    @pl.when(blk >= 2)
                    def _():
                        wait_write(blk - 2, b)

                    transpose_cols(b, TBLK)
                    start_write(blk, b)

                    @pl.when(blk + 2 < nblk_w)
                    def _():
                        start_in(blk + 2, b)

            return carry

        lax.fori_loop(0, (nblk_w + 1) // 2, step, 0)

        @pl.when(nblk_w >= 2)
        def _():
            wait_write(nblk_w - 2, (nblk_w - 2) % 2)

        @pl.when(nblk_w >= 1)
        def _():
            wait_write(nblk_w - 1, (nblk_w - 1) % 2)

        @pl.when(wid == 0)
        def _():
            pltpu.sync_copy(
                wt_t.at[:, pl.ds(NFULL * TBLK, TAIL)],
                in_v.at[0, :, pl.ds(0, TAIL)])
            transpose_cols(0, TAIL)
            pltpu.sync_copy(
                out_v.at[0, pl.ds(0, TAIL), :],
                wt_rm.at[pl.ds(NFULL * TBLK, TAIL), :])

    return transpose_kernel


def _make_gather(num_cores, num_subcores):
    nw = num_cores * num_subcores
    total_chunks = ROWS // CHUNK
    chunks_per_w = total_chunks // nw
    chunks_per_s = B_TOK // CHUNK

    mesh = plsc.VectorSubcoreMesh(
        core_axis_name="c",
        subcore_axis_name="s",
        num_cores=num_cores,
        num_subcores=num_subcores,
    )

    @functools.partial(
        pl.kernel,
        out_type=jax.ShapeDtypeStruct(
            (S_TOK, DIM // 8, B_TOK // CHUNK, 8 * CHUNK), jnp.float32),
        mesh=mesh,
        scratch_types=[
            pltpu.VMEM((NBUF, CHUNK), jnp.int32),
            pltpu.VMEM((NBUF, CHUNK, DIM), jnp.float32),
            pltpu.VMEM((NBUF, DIM // 8, 8 * CHUNK), jnp.float32),
            pltpu.SemaphoreType.DMA((NBUF,)),
            pltpu.SemaphoreType.DMA((NBUF,)),
            pltpu.SemaphoreType.DMA((NBUF,)),
        ],
        compiler_params=_PARAMS,
    )
    def gather_kernel(idx_hbm, table_hbm, out_hbm, idx_v, rows_v, tr_v,
                      isem, gsem, wsem):
        wid = lax.axis_index("s") * num_cores + lax.axis_index("c")
        chunk0 = wid * chunks_per_w
        iota16 = lax.iota(jnp.int32, 16)

        def coords(chunk_j):
            cg = chunk0 + chunk_j
            return cg // chunks_per_s, cg % chunks_per_s

        def idx_src(chunk_j):
            s_id, bj = coords(chunk_j)
            return idx_hbm.at[s_id, pl.ds(bj * CHUNK, CHUNK)]

        def start_idx(chunk_j, buf):
            pltpu.async_copy(idx_src(chunk_j), idx_v.at[buf], isem.at[buf])

        def wait_idx(chunk_j, buf):
            pltpu.make_async_copy(
                idx_src(chunk_j), idx_v.at[buf], isem.at[buf]).wait()

        def start_gather(buf):
            pltpu.async_copy(
                table_hbm.at[idx_v.at[buf]], rows_v.at[buf], gsem.at[buf])

        def wait_gather(buf):
            pltpu.make_async_copy(
                table_hbm.at[idx_v.at[buf]], rows_v.at[buf],
                gsem.at[buf]).wait()

        def out_dst(chunk_j):
            s_id, bj = coords(chunk_j)
            return out_hbm.at[s_id, :, bj, :]

        def start_write(chunk_j, buf):
            pltpu.async_copy(tr_v.at[buf], out_dst(chunk_j), wsem.at[buf])

        def wait_write(chunk_j, buf):
            pltpu.make_async_copy(
                tr_v.at[buf], out_dst(chunk_j), wsem.at[buf]).wait()

        def transpose_chunk(buf):
            def body(g, carry):
                cvec = g * 16 + iota16
                for d in range(DIM):
                    dvec = jnp.broadcast_to(jnp.int32(d), (16,))
                    vals = plsc.load_gather(rows_v.at[buf], [cvec, dvec])
                    tr_v[buf, d // 8,
                         pl.ds((d % 8) * CHUNK + g * 16, 16)] = vals
                return carry

            lax.fori_loop(0, CHUNK // 16, body, 0)

        for b in range(NBUF):
            start_idx(b, b)

        def step(i, carry):
            cbase = i * NBUF
            for b in range(NBUF):
                c = cbase + b

                @pl.when(c < chunks_per_w)
                def _():
                    wait_idx(c, b)
                    start_gather(b)

                r = c - LAG
                br = (b - LAG) % NBUF

                @pl.when((r >= 0) & (r < chunks_per_w))
                def _():
                    wait_gather(br)

                    @pl.when(r >= NBUF)
                    def _():
                        wait_write(r - NBUF, br)

                    transpose_chunk(br)
                    start_write(r, br)

                    @pl.when(r + NBUF < chunks_per_w)
                    def _():
                        start_idx(r + NBUF, br)

            return carry

        lax.fori_loop(0, (chunks_per_w + LAG + NBUF - 1) // NBUF, step, 0)

        for k in range(NBUF):
            r = chunks_per_w - NBUF + k
            wait_write(r, r % NBUF)

    return gather_kernel


def kernel(token_ids, weight):
    num_cores, num_subcores = _sc_geometry()
    idx_t = token_ids.T.astype(jnp.int32)
    out4 = _make_gather(num_cores, num_subcores)(idx_t, weight)
    out5 = out4.reshape(S_TOK, DIM // 8, B_TOK // CHUNK, 8, CHUNK)
    return out5.transpose(2, 4, 0, 1, 3).reshape(B_TOK, S_TOK, DIM)


# speedup vs baseline: 8.0493x; 1.4057x over previous
import functools

import jax
import jax.numpy as jnp
from jax import lax
from jax.experimental import pallas as pl
from jax.experimental.pallas import tpu as pltpu
from jax.experimental.pallas import tpu_sc as plsc

B_TOK = 16384
S_TOK = 26
ROWS = B_TOK * S_TOK
DIM = 64
CHUNK = 128
NBUF = 4
LAG = 2


def _sc_geometry():
    try:
        info = plsc.get_sparse_core_info()
        return info.num_cores, info.num_subcores
    except Exception:
        return 2, 16


def _make_sc_gather(num_cores, num_subcores):
    nw = num_cores * num_subcores
    total_chunks = ROWS // CHUNK
    chunks_per_w = total_chunks // nw
    chunks_per_s = B_TOK // CHUNK

    mesh = plsc.VectorSubcoreMesh(
        core_axis_name="c",
        subcore_axis_name="s",
        num_cores=num_cores,
        num_subcores=num_subcores,
    )

    @functools.partial(
        pl.kernel,
        out_type=jax.ShapeDtypeStruct((S_TOK, B_TOK, DIM), jnp.float32),
        mesh=mesh,
        scratch_types=[
            pltpu.VMEM((NBUF, CHUNK), jnp.int32),
            pltpu.VMEM((NBUF, CHUNK, DIM), jnp.float32),
            pltpu.SemaphoreType.DMA((NBUF,)),
            pltpu.SemaphoreType.DMA((NBUF,)),
            pltpu.SemaphoreType.DMA((NBUF,)),
        ],
        compiler_params=pltpu.CompilerParams(use_tc_tiling_on_sc=False),
    )
    def gather_kernel(idx_hbm, table_hbm, out_hbm, idx_v, rows_v,
                      isem, gsem, wsem):
        wid = lax.axis_index("s") * num_cores + lax.axis_index("c")
        chunk0 = wid * chunks_per_w

        def coords(chunk_j):
            cg = chunk0 + chunk_j
            return cg // chunks_per_s, (cg % chunks_per_s) * CHUNK

        def idx_src(chunk_j):
            s_id, b0 = coords(chunk_j)
            return idx_hbm.at[s_id, pl.ds(b0, CHUNK)]

        def start_idx(chunk_j, buf):
            pltpu.async_copy(idx_src(chunk_j), idx_v.at[buf], isem.at[buf])

        def wait_idx(chunk_j, buf):
            pltpu.make_async_copy(
                idx_src(chunk_j), idx_v.at[buf], isem.at[buf]).wait()

        def start_gather(buf):
            pltpu.async_copy(
                table_hbm.at[idx_v.at[buf]], rows_v.at[buf], gsem.at[buf])

        def wait_gather(buf):
            pltpu.make_async_copy(
                table_hbm.at[idx_v.at[buf]], rows_v.at[buf],
                gsem.at[buf]).wait()

        def out_dst(chunk_j):
            s_id, b0 = coords(chunk_j)
            return out_hbm.at[s_id, pl.ds(b0, CHUNK), :]

        def start_write(chunk_j, buf):
            pltpu.async_copy(rows_v.at[buf], out_dst(chunk_j), wsem.at[buf])

        def wait_write(chunk_j, buf):
            pltpu.make_async_copy(
                rows_v.at[buf], out_dst(chunk_j), wsem.at[buf]).wait()

        for b in range(NBUF):
            start_idx(b, b)

        def step(i, carry):
            cbase = i * NBUF
            for b in range(NBUF):
                c = cbase + b

                @pl.when(c < chunks_per_w)
                def _():
                    wait_idx(c, b)

                    @pl.when(c >= NBUF)
                    def _():
                        wait_write(c - NBUF, b)

                    start_gather(b)

                r = c - LAG
                br = (b - LAG) % NBUF

                @pl.when((r >= 0) & (r < chunks_per_w))
                def _():
                    wait_gather(br)
                    start_write(r, br)

                    @pl.when(r + NBUF < chunks_per_w)
                    def _():
                        start_idx(r + NBUF, br)

            return carry

        lax.fori_loop(0, (chunks_per_w + LAG + NBUF - 1) // NBUF, step, 0)

        for b in range(NBUF):
            wait_write(chunks_per_w - NBUF + b,
                       (chunks_per_w - NBUF + b) % NBUF)

    return gather_kernel


def kernel(token_ids, weight):
    num_cores, num_subcores = _sc_geometry()
    idx_t = token_ids.T.astype(jnp.int32)
    out3 = _make_sc_gather(num_cores, num_subcores)(idx_t, weight)
    return out3.transpose(1, 0, 2)


# speedup vs baseline: 8.2698x; 1.0274x over previous
import functools

import jax
import jax.numpy as jnp
from jax import lax
from jax.experimental import pallas as pl
from jax.experimental.pallas import tpu as pltpu
from jax.experimental.pallas import tpu_sc as plsc

B_TOK = 16384
S_TOK = 26
ROWS = B_TOK * S_TOK
DIM = 64
CHUNK = 128
NBUF = 4
LAG = 2


def _sc_geometry():
    try:
        info = plsc.get_sparse_core_info()
        return info.num_cores, info.num_subcores
    except Exception:
        return 2, 16


def _make_sc_gather(num_cores, num_subcores):
    nw = num_cores * num_subcores
    total_chunks = ROWS // CHUNK
    chunks_per_w = total_chunks // nw
    chunks_per_s = B_TOK // CHUNK

    mesh = plsc.VectorSubcoreMesh(
        core_axis_name="c",
        subcore_axis_name="s",
        num_cores=num_cores,
        num_subcores=num_subcores,
    )

    @functools.partial(
        pl.kernel,
        out_type=jax.ShapeDtypeStruct((S_TOK, B_TOK, DIM), jnp.float32),
        mesh=mesh,
        scratch_types=[
            pltpu.VMEM((NBUF, CHUNK), jnp.int32),
            pltpu.VMEM((NBUF, CHUNK, 2 * DIM), jnp.float32),
            pltpu.SemaphoreType.DMA((NBUF,)),
            pltpu.SemaphoreType.DMA((NBUF,)),
            pltpu.SemaphoreType.DMA((NBUF,)),
        ],
        compiler_params=pltpu.CompilerParams(use_tc_tiling_on_sc=False),
    )
    def gather_kernel(idx_hbm, table_hbm, out_hbm, idx_v, rows_v,
                      isem, gsem, wsem):
        wid = lax.axis_index("s") * num_cores + lax.axis_index("c")
        chunk0 = wid * chunks_per_w

        def coords(chunk_j):
            cg = chunk0 + chunk_j
            return cg // chunks_per_s, (cg % chunks_per_s) * CHUNK

        def idx_src(chunk_j):
            s_id, b0 = coords(chunk_j)
            return idx_hbm.at[s_id, pl.ds(b0, CHUNK)]

        def start_idx(chunk_j, buf):
            pltpu.async_copy(idx_src(chunk_j), idx_v.at[buf], isem.at[buf])

        def wait_idx(chunk_j, buf):
            pltpu.make_async_copy(
                idx_src(chunk_j), idx_v.at[buf], isem.at[buf]).wait()

        def start_gather(buf):
            pltpu.async_copy(
                table_hbm.at[idx_v.at[buf]], rows_v.at[buf], gsem.at[buf])

        def wait_gather(buf):
            pltpu.make_async_copy(
                table_hbm.at[idx_v.at[buf]], rows_v.at[buf],
                gsem.at[buf]).wait()

        def out_dst(chunk_j):
            s_id, b0 = coords(chunk_j)
            return out_hbm.at[s_id, pl.ds(b0, CHUNK), :]

        def start_write(chunk_j, buf):
            pltpu.async_copy(
                rows_v.at[buf, :, pl.ds(0, DIM)], out_dst(chunk_j),
                wsem.at[buf])

        def wait_write(chunk_j, buf):
            pltpu.make_async_copy(
                rows_v.at[buf, :, pl.ds(0, DIM)], out_dst(chunk_j),
                wsem.at[buf]).wait()

        for b in range(NBUF):
            start_idx(b, b)

        def step(i, carry):
            cbase = i * NBUF
            for b in range(NBUF):
                c = cbase + b

                @pl.when(c < chunks_per_w)
                def _():
                    wait_idx(c, b)

                    @pl.when(c >= NBUF)
                    def _():
                        wait_write(c - NBUF, b)

                    start_gather(b)

                r = c - LAG
                br = (b - LAG) % NBUF

                @pl.when((r >= 0) & (r < chunks_per_w))
                def _():
                    wait_gather(br)
                    start_write(r, br)

                    @pl.when(r + NBUF < chunks_per_w)
                    def _():
                        start_idx(r + NBUF, br)

            return carry

        lax.fori_loop(0, (chunks_per_w + LAG + NBUF - 1) // NBUF, step, 0)

        for b in range(NBUF):
            wait_write(chunks_per_w - NBUF + b,
                       (chunks_per_w - NBUF + b) % NBUF)

    return gather_kernel


def kernel(token_ids, weight):
    num_cores, num_subcores = _sc_geometry()
    idx_t = token_ids.T.astype(jnp.int32)
    wt_pad = jnp.pad(weight, ((0, 0), (0, DIM)))
    out3 = _make_sc_gather(num_cores, num_subcores)(idx_t, wt_pad)
    return out3.transpose(1, 0, 2)


# speedup vs baseline: 8.8161x; 1.0661x over previous
import functools

import jax
import jax.numpy as jnp
from jax import lax
from jax.experimental import pallas as pl
from jax.experimental.pallas import tpu as pltpu
from jax.experimental.pallas import tpu_sc as plsc

B_TOK = 16384
S_TOK = 26
ROWS = B_TOK * S_TOK
DIM = 64
CHUNK = 128
NBUF = 4
LAG = 2


def _sc_geometry():
    try:
        info = plsc.get_sparse_core_info()
        return info.num_cores, info.num_subcores
    except Exception:
        return 2, 16


def _make_sc_gather(num_cores, num_subcores):
    nw = num_cores * num_subcores
    total_chunks = ROWS // CHUNK
    chunks_per_w = total_chunks // nw
    chunks_per_s = B_TOK // CHUNK

    mesh = plsc.VectorSubcoreMesh(
        core_axis_name="c",
        subcore_axis_name="s",
        num_cores=num_cores,
        num_subcores=num_subcores,
    )

    @functools.partial(
        pl.kernel,
        out_type=jax.ShapeDtypeStruct(
            (S_TOK, DIM // 8, B_TOK // CHUNK, 8 * CHUNK), jnp.float32),
        mesh=mesh,
        scratch_types=[
            pltpu.VMEM((NBUF, CHUNK), jnp.int32),
            pltpu.VMEM((NBUF, CHUNK, 2 * DIM), jnp.float32),
            pltpu.VMEM((NBUF, DIM // 8, 8 * CHUNK), jnp.float32),
            pltpu.SemaphoreType.DMA((NBUF,)),
            pltpu.SemaphoreType.DMA((NBUF,)),
            pltpu.SemaphoreType.DMA((NBUF,)),
        ],
        compiler_params=pltpu.CompilerParams(
            use_tc_tiling_on_sc=False, needs_layout_passes=False),
    )
    def gather_kernel(idx_hbm, table_hbm, out_hbm, idx_v, rows_v, tr_v,
                      isem, gsem, wsem):
        wid = lax.axis_index("s") * num_cores + lax.axis_index("c")
        chunk0 = wid * chunks_per_w
        iota16 = lax.iota(jnp.int32, 16)

        def coords(chunk_j):
            cg = chunk0 + chunk_j
            return cg // chunks_per_s, cg % chunks_per_s

        def idx_src(chunk_j):
            s_id, bj = coords(chunk_j)
            return idx_hbm.at[s_id, pl.ds(bj * CHUNK, CHUNK)]

        def start_idx(chunk_j, buf):
            pltpu.async_copy(idx_src(chunk_j), idx_v.at[buf], isem.at[buf])

        def wait_idx(chunk_j, buf):
            pltpu.make_async_copy(
                idx_src(chunk_j), idx_v.at[buf], isem.at[buf]).wait()

        def start_gather(buf):
            pltpu.async_copy(
                table_hbm.at[idx_v.at[buf]], rows_v.at[buf], gsem.at[buf])

        def wait_gather(buf):
            pltpu.make_async_copy(
                table_hbm.at[idx_v.at[buf]], rows_v.at[buf],
                gsem.at[buf]).wait()

        def out_dst(chunk_j):
            s_id, bj = coords(chunk_j)
            return out_hbm.at[s_id, :, bj, :]

        def start_write(chunk_j, buf):
            pltpu.async_copy(tr_v.at[buf], out_dst(chunk_j), wsem.at[buf])

        def wait_write(chunk_j, buf):
            pltpu.make_async_copy(
                tr_v.at[buf], out_dst(chunk_j), wsem.at[buf]).wait()

        def transpose_chunk(buf):
            def body(g, carry):
                cvec = g * 16 + iota16
                for d in range(DIM):
                    dpvec = lax.bitwise_and(iota16 + d, jnp.int32(DIM - 1))
                    vals = plsc.load_gather(rows_v.at[buf], [cvec, dpvec])
                    plsc.store_scatter(
                        tr_v.at[buf],
                        [lax.shift_right_logical(dpvec, 3),
                         lax.shift_left(
                             lax.bitwise_and(dpvec, jnp.int32(7)), 7)
                         + cvec],
                        vals)
                return carry

            lax.fori_loop(0, CHUNK // 16, body, 0)

        for b in range(NBUF):
            start_idx(b, b)

        def step(i, carry):
            cbase = i * NBUF
            for b in range(NBUF):
                c = cbase + b

                @pl.when(c < chunks_per_w)
                def _():
                    wait_idx(c, b)
                    start_gather(b)

                r = c - LAG
                br = (b - LAG) % NBUF

                @pl.when((r >= 0) & (r < chunks_per_w))
                def _():
                    wait_gather(br)

                    @pl.when(r >= NBUF)
                    def _():
                        wait_write(r - NBUF, br)

                    transpose_chunk(br)
                    start_write(r, br)

                    @pl.when(r + NBUF < chunks_per_w)
                    def _():
                        start_idx(r + NBUF, br)

            return carry

        lax.fori_loop(0, (chunks_per_w + LAG + NBUF - 1) // NBUF, step, 0)

        for b in range(NBUF):
            wait_write(chunks_per_w - NBUF + b,
                       (chunks_per_w - NBUF + b) % NBUF)

    return gather_kernel


def kernel(token_ids, weight):
    num_cores, num_subcores = _sc_geometry()
    idx_t = token_ids.T.astype(jnp.int32)
    wt_pad = jnp.pad(weight, ((0, 0), (0, DIM)))
    out4 = _make_sc_gather(num_cores, num_subcores)(idx_t, wt_pad)
    out5 = out4.reshape(S_TOK, DIM // 8, B_TOK // CHUNK, 8, CHUNK)
    return out5.transpose(2, 4, 0, 1, 3).reshape(B_TOK, S_TOK, DIM)
